# Initial kernel scaffold; baseline (speedup 1.0000x reference)
import jax
import jax.numpy as jnp
from jax.experimental import pallas as pl


def kernel(x, mapping, table):
    raise NotImplementedError("write your pallas kernel here")


# speedup vs baseline: 4.9641x; 4.9641x over previous
import functools

import jax
import jax.numpy as jnp
from jax import lax
from jax.experimental import pallas as pl
from jax.experimental.pallas import tpu as pltpu
from jax.experimental.pallas import tpu_sc as plsc

BATCH = 1024
INPUT_SIZE = 2048
OUTPUT_SIZE = 2048
N_INPUTS = 4
NUM_COMBOS = 16
LANES = 16

NUM_CORES = 2
NUM_SUBCORES = 16
NUM_WORKERS = NUM_CORES * NUM_SUBCORES
ROWS_PER_WORKER = BATCH // NUM_WORKERS
GROUPS_PER_SLAB = 8
SLAB = LANES * GROUPS_PER_SLAB
NUM_SLABS = OUTPUT_SIZE // SLAB

_MESH = plsc.VectorSubcoreMesh(core_axis_name="c", subcore_axis_name="s")


@functools.partial(
    pl.kernel,
    mesh=_MESH,
    compiler_params=pltpu.CompilerParams(
        use_tc_tiling_on_sc=False, needs_layout_passes=False),
    out_type=jax.ShapeDtypeStruct((BATCH, OUTPUT_SIZE), jnp.float32),
    scratch_types=[
        pltpu.VMEM((ROWS_PER_WORKER * INPUT_SIZE,), jnp.float32),
        pltpu.VMEM((NUM_COMBOS, OUTPUT_SIZE), jnp.float32),
        pltpu.VMEM((N_INPUTS, OUTPUT_SIZE), jnp.int32),
        pltpu.VMEM((ROWS_PER_WORKER, SLAB), jnp.float32),
    ],
)
def _lut_sc(x_hbm, tab_hbm, map_hbm, out_hbm, x_v, tab_v, map_v, ostage_v):
    wid = lax.axis_index("s") * NUM_CORES + lax.axis_index("c")
    row_base = wid * ROWS_PER_WORKER
    pltpu.sync_copy(
        x_hbm.at[pl.ds(row_base * INPUT_SIZE, ROWS_PER_WORKER * INPUT_SIZE)],
        x_v)
    pltpu.sync_copy(tab_hbm, tab_v)
    pltpu.sync_copy(map_hbm, map_v)

    def slab_body(gg, carry):
        cbase = gg * SLAB
        for gi in range(GROUPS_PER_SLAB):
            nbase = cbase + gi * LANES
            c = [tab_v[s, pl.ds(nbase, LANES)] for s in range(NUM_COMBOS)]
            for maskb in (1, 2, 4, 8):
                for s in range(NUM_COMBOS):
                    if s & maskb:
                        c[s] = c[s] - c[s ^ maskb]
            midx = [map_v[k, pl.ds(nbase, LANES)] for k in range(N_INPUTS)]

            def row_body(r, rcarry, c=c, midx=midx, gi=gi):
                roff = r * INPUT_SIZE
                m = [plsc.load_gather(x_v, [midx[k] + roff])
                     for k in range(N_INPUTS)]
                h = {s: c[s] for s in range(NUM_COMBOS)}
                for maskb, kbit in ((8, 3), (4, 2), (2, 1), (1, 0)):
                    h = {s: h[s] + m[kbit] * h[s | maskb]
                         for s in h if not s & maskb}
                ostage_v[r, pl.ds(gi * LANES, LANES)] = h[0]
                return rcarry

            lax.fori_loop(0, ROWS_PER_WORKER, row_body, 0)
        pltpu.sync_copy(
            ostage_v,
            out_hbm.at[pl.ds(row_base, ROWS_PER_WORKER), pl.ds(cbase, SLAB)],
        )
        return carry

    lax.fori_loop(0, NUM_SLABS, slab_body, 0)


def kernel(x, mapping, table):
    x_flat = x.reshape(-1)
    tab_t = table.T.astype(jnp.float32)
    map_t = mapping.T.astype(jnp.int32)
    return _lut_sc(x_flat, tab_t, map_t)
